# baseline (speedup 1.0000x reference)
import functools

import jax
import jax.numpy as jnp
from jax import lax
from jax.experimental import pallas as pl
from jax.experimental.pallas import tpu as pltpu
from jax.experimental.pallas import tpu_sc as plsc

WORD_VOCAB = 100000
TOPIC_VOCAB = 1000
ACT_VOCAB = 1000
WORD_D = 64
TOPIC_D = 32
ACT_D = 32
B = 4096
L = 200

_NC = 2
_NS = 16
_NW = _NC * _NS

_TOTAL_W = B * L
_W_PER = _TOTAL_W // _NW
_SUPER = 128
_NSUPER = _W_PER // _SUPER
_S_PER = B // _NW
_LANES = 16
_RB = 64


def _body(word_ids, topic_ids, act_ids, over_w, over_t, over_a,
          word_out, topic_out, act_out,
          g_a, g_b, t_a, t_b, i_a, i_b, tidx_v, aidx_v, tt_v, at_v,
          gsem_a, gsem_b, osem_a, osem_b, isem_a, isem_b, tsem):
  c = lax.axis_index("c")
  s = lax.axis_index("s")
  wid = s * _NC + c

  idx_base = wid * _W_PER
  out_base = wid * (_W_PER // 2)
  orows = _SUPER // 2

  def iload(i, ib, isem):
    pltpu.async_copy(word_ids.at[pl.ds(idx_base + i * _SUPER, _SUPER)],
                     ib, isem)

  def iwait(ib, isem):
    pltpu.make_async_copy(word_ids.at[pl.ds(0, _SUPER)], ib, isem).wait()

  def fire(gb, ib, gs):
    pltpu.async_copy(over_w.at[ib], gb, gs)

  def gwait(gb, ib, gs):
    pltpu.make_async_copy(over_w.at[ib], gb, gs).wait()

  def compact_w(gb, tb):
    @plsc.parallel_loop(0, _SUPER // 2, unroll=8)
    def _cp(q):
      g8 = ((q >> 2) << 3) + (q & 3)
      for cc in range(4):
        tb[q, pl.ds(cc * _LANES, _LANES)] = gb[g8, pl.ds(cc * _LANES, _LANES)]
      for cc in range(4):
        tb[q, pl.ds(64 + cc * _LANES, _LANES)] = (
            gb[g8 + 4, pl.ds(cc * _LANES, _LANES)])

  def compact_s(gb, tb):
    @plsc.parallel_loop(0, _S_PER // 4, unroll=8)
    def _cp(q):
      for h in range(4):
        for cc in range(2):
          tb[q, pl.ds(h * 32 + cc * _LANES, _LANES)] = (
              gb[4 * q + h, pl.ds(cc * _LANES, _LANES)])

  def out_start(i, tb, os):
    pltpu.async_copy(tb, word_out.at[pl.ds(out_base + i * orows, orows)], os)

  def out_wait(tb, os):
    pltpu.make_async_copy(tb, word_out.at[pl.ds(out_base, orows)], os).wait()

  slots = ((g_a, t_a, i_a, gsem_a, osem_a, isem_a),
           (g_b, t_b, i_b, gsem_b, osem_b, isem_b))

  for i, (gb, tb, ib, gs, os, isem) in ((0, slots[0]), (1, slots[1])):
    iload(i, ib, isem)
  for i, (gb, tb, ib, gs, os, isem) in ((0, slots[0]), (1, slots[1])):
    iwait(ib, isem)
    fire(gb, ib, gs)
  for i, (gb, tb, ib, gs, os, isem) in ((0, slots[0]), (1, slots[1])):
    gwait(gb, ib, gs)
    iload(i + 2, ib, isem)
    compact_w(gb, tb)
    iwait(ib, isem)
    fire(gb, ib, gs)
    out_start(i, tb, os)

  @pl.loop(1, _NSUPER // 2 - 1)
  def _w(ci):
    for slot, (gb, tb, ib, gs, os, isem) in enumerate(slots):
      i = ci * 2 + slot
      gwait(gb, ib, gs)
      iload(i + 2, ib, isem)
      out_wait(tb, os)
      compact_w(gb, tb)
      iwait(ib, isem)
      fire(gb, ib, gs)
      out_start(i, tb, os)

  for i, (gb, tb, ib, gs, os, isem) in ((_NSUPER - 2, slots[0]),
                                        (_NSUPER - 1, slots[1])):
    gwait(gb, ib, gs)
    out_wait(tb, os)
    compact_w(gb, tb)
    out_start(i, tb, os)
  for gb, tb, ib, gs, os, isem in slots:
    out_wait(tb, os)

  sb = wid * _S_PER
  pltpu.sync_copy(topic_ids.at[pl.ds(sb, _S_PER)], tidx_v)
  pltpu.sync_copy(act_ids.at[pl.ds(sb, _S_PER)], aidx_v)
  dt = pltpu.async_copy(over_t.at[tidx_v], g_a, tsem)
  da = pltpu.async_copy(over_a.at[aidx_v], g_b, tsem)
  dt.wait()
  da.wait()
  sorows = _S_PER // 4
  compact_s(g_a, tt_v)
  compact_s(g_b, at_v)
  pltpu.sync_copy(tt_v, topic_out.at[pl.ds(wid * sorows, sorows)])
  pltpu.sync_copy(at_v, act_out.at[pl.ds(wid * sorows, sorows)])


@jax.jit
def _run(word_ids1d, topic_ids, act_ids, over_w, over_t, over_a):
  mesh = plsc.VectorSubcoreMesh(core_axis_name="c", subcore_axis_name="s")
  k = pl.kernel(
      _body,
      out_type=(
          jax.ShapeDtypeStruct((_TOTAL_W // 2, 128), jnp.float32),
          jax.ShapeDtypeStruct((B // 4, 128), jnp.float32),
          jax.ShapeDtypeStruct((B // 4, 128), jnp.float32),
      ),
      mesh=mesh,
      scratch_types=(
          pltpu.VMEM((_SUPER, 128), jnp.float32),
          pltpu.VMEM((_SUPER, 128), jnp.float32),
          pltpu.VMEM((_SUPER // 2, 128), jnp.float32),
          pltpu.VMEM((_SUPER // 2, 128), jnp.float32),
          pltpu.VMEM((_SUPER,), jnp.int32),
          pltpu.VMEM((_SUPER,), jnp.int32),
          pltpu.VMEM((_S_PER,), jnp.int32),
          pltpu.VMEM((_S_PER,), jnp.int32),
          pltpu.VMEM((_S_PER // 4, 128), jnp.float32),
          pltpu.VMEM((_S_PER // 4, 128), jnp.float32),
          pltpu.SemaphoreType.DMA,
          pltpu.SemaphoreType.DMA,
          pltpu.SemaphoreType.DMA,
          pltpu.SemaphoreType.DMA,
          pltpu.SemaphoreType.DMA,
          pltpu.SemaphoreType.DMA,
          pltpu.SemaphoreType.DMA,
      ),
  )
  wout128, tout128, aout128 = k(word_ids1d, topic_ids, act_ids,
                                over_w, over_t, over_a)

  def _repack_body(x_ref, o_ref):
    x = x_ref[...]
    left = x[:, :WORD_D].reshape(-1, 4, WORD_D)
    right = x[:, WORD_D:].reshape(-1, 4, WORD_D)
    o_ref[...] = jnp.concatenate([left, right], axis=1).reshape(o_ref.shape)

  wout = pl.pallas_call(
      _repack_body,
      grid=(B // _RB,),
      in_specs=[pl.BlockSpec((_RB * L // 2, 128), lambda i: (i, 0))],
      out_specs=pl.BlockSpec((_RB, L, WORD_D), lambda i: (i, 0, 0)),
      out_shape=jax.ShapeDtypeStruct((B, L, WORD_D), jnp.float32),
  )(wout128)

  def _repack_small(x128, d):
    def body(x_ref, o_ref):
      x = x_ref[...]
      parts = [x[:, h * d:(h + 1) * d].reshape(-1, 1, d)
               for h in range(128 // d)]
      o_ref[...] = jnp.concatenate(parts, axis=1).reshape(o_ref.shape)
    return pl.pallas_call(
        body,
        in_specs=[pl.BlockSpec((B // 4, 128), lambda: (0, 0))],
        out_specs=pl.BlockSpec((B, d), lambda: (0, 0)),
        out_shape=jax.ShapeDtypeStruct((B, d), jnp.float32),
    )(x128)

  tout = _repack_small(tout128, TOPIC_D)
  aout = _repack_small(aout128, ACT_D)
  return wout, tout, aout


def _overlap(table, d):
  return jnp.concatenate([table] * (128 // d), axis=1)


def kernel(word_ids, topic_ids, act_ids, word_table, topic_table, act_table):
  word_ids1d = word_ids.reshape(_TOTAL_W).astype(jnp.int32)
  return _run(word_ids1d,
              topic_ids.astype(jnp.int32),
              act_ids.astype(jnp.int32),
              _overlap(word_table, WORD_D),
              _overlap(topic_table, TOPIC_D),
              _overlap(act_table, ACT_D))


# speedup vs baseline: 1.4729x; 1.4729x over previous
import functools

import jax
import jax.numpy as jnp
from jax import lax
from jax.experimental import pallas as pl
from jax.experimental.pallas import tpu as pltpu
from jax.experimental.pallas import tpu_sc as plsc

WORD_VOCAB = 100000
TOPIC_VOCAB = 1000
ACT_VOCAB = 1000
WORD_D = 64
TOPIC_D = 32
ACT_D = 32
B = 4096
L = 200

_NC = 2
_NS = 16
_NW = _NC * _NS

_TOTAL_W = B * L
_W_PER = _TOTAL_W // _NW
_SUPER = 128
_NSUPER = _W_PER // _SUPER
_S_PER = B // _NW
_LANES = 16
_WPACK = 128 // WORD_D
_SPACK = 128 // TOPIC_D


def _body(word_ids, topic_ids, act_ids, over_w, over_t, over_a,
          word_out, topic_out, act_out,
          g_a, g_b, t_a, t_b, i_a, i_b, tidx_v, aidx_v, tt_v, at_v,
          gsem_a, gsem_b, osem_a, osem_b, isem_a, isem_b, tsem):
  c = lax.axis_index("c")
  s = lax.axis_index("s")
  wid = s * _NC + c

  idx_base = wid * _W_PER
  out_base = wid * _W_PER
  orows = _SUPER

  def iload(i, ib, isem):
    pltpu.async_copy(word_ids.at[pl.ds(idx_base + i * _SUPER, _SUPER)],
                     ib, isem)

  def iwait(ib, isem):
    pltpu.make_async_copy(word_ids.at[pl.ds(0, _SUPER)], ib, isem).wait()

  def fire(gb, ib, gs):
    pltpu.async_copy(over_w.at[ib], gb, gs)

  def gwait(gb, ib, gs):
    pltpu.make_async_copy(over_w.at[ib], gb, gs).wait()

  def compact(gb, tb, d, nrows, unroll):
    nv = d // _LANES
    @plsc.parallel_loop(0, nrows, unroll=unroll)
    def _cp(q):
      for cc in range(nv):
        tb[q, pl.ds(cc * _LANES, _LANES)] = gb[q, pl.ds(cc * _LANES, _LANES)]

  def out_start(i, tb, os):
    pltpu.async_copy(tb, word_out.at[pl.ds(out_base + i * orows, orows)], os)

  def out_wait(tb, os):
    pltpu.make_async_copy(tb, word_out.at[pl.ds(out_base, orows)], os).wait()

  slots = ((g_a, t_a, i_a, gsem_a, osem_a, isem_a),
           (g_b, t_b, i_b, gsem_b, osem_b, isem_b))

  for i, (gb, tb, ib, gs, os, isem) in ((0, slots[0]), (1, slots[1])):
    iload(i, ib, isem)
  for i, (gb, tb, ib, gs, os, isem) in ((0, slots[0]), (1, slots[1])):
    iwait(ib, isem)
    fire(gb, ib, gs)
  for i, (gb, tb, ib, gs, os, isem) in ((0, slots[0]), (1, slots[1])):
    gwait(gb, ib, gs)
    iload(i + 2, ib, isem)
    compact(gb, tb, WORD_D, orows, 16)
    iwait(ib, isem)
    fire(gb, ib, gs)
    out_start(i, tb, os)

  @pl.loop(1, _NSUPER // 2 - 1)
  def _w(ci):
    for slot, (gb, tb, ib, gs, os, isem) in enumerate(slots):
      i = ci * 2 + slot
      gwait(gb, ib, gs)
      iload(i + 2, ib, isem)
      out_wait(tb, os)
      compact(gb, tb, WORD_D, orows, 16)
      iwait(ib, isem)
      fire(gb, ib, gs)
      out_start(i, tb, os)

  for i, (gb, tb, ib, gs, os, isem) in ((_NSUPER - 2, slots[0]),
                                        (_NSUPER - 1, slots[1])):
    gwait(gb, ib, gs)
    out_wait(tb, os)
    compact(gb, tb, WORD_D, orows, 16)
    out_start(i, tb, os)
  for gb, tb, ib, gs, os, isem in slots:
    out_wait(tb, os)

  sb = wid * _S_PER
  pltpu.sync_copy(topic_ids.at[pl.ds(sb, _S_PER)], tidx_v)
  pltpu.sync_copy(act_ids.at[pl.ds(sb, _S_PER)], aidx_v)
  dt = pltpu.async_copy(over_t.at[tidx_v], g_a, tsem)
  da = pltpu.async_copy(over_a.at[aidx_v], g_b, tsem)
  dt.wait()
  da.wait()
  compact(g_a, tt_v, TOPIC_D, _S_PER, 16)
  compact(g_b, at_v, ACT_D, _S_PER, 16)
  pltpu.sync_copy(tt_v, topic_out.at[pl.ds(sb, _S_PER)])
  pltpu.sync_copy(at_v, act_out.at[pl.ds(sb, _S_PER)])


@jax.jit
def _run(word_ids1d, topic_ids, act_ids, over_w, over_t, over_a):
  mesh = plsc.VectorSubcoreMesh(core_axis_name="c", subcore_axis_name="s")
  k = pl.kernel(
      _body,
      out_type=(
          jax.ShapeDtypeStruct((_TOTAL_W, WORD_D), jnp.float32),
          jax.ShapeDtypeStruct((B, TOPIC_D), jnp.float32),
          jax.ShapeDtypeStruct((B, ACT_D), jnp.float32),
      ),
      mesh=mesh,
      scratch_types=(
          pltpu.VMEM((_SUPER, 128), jnp.float32),
          pltpu.VMEM((_SUPER, 128), jnp.float32),
          pltpu.VMEM((_SUPER, WORD_D), jnp.float32),
          pltpu.VMEM((_SUPER, WORD_D), jnp.float32),
          pltpu.VMEM((_SUPER,), jnp.int32),
          pltpu.VMEM((_SUPER,), jnp.int32),
          pltpu.VMEM((_S_PER,), jnp.int32),
          pltpu.VMEM((_S_PER,), jnp.int32),
          pltpu.VMEM((_S_PER, TOPIC_D), jnp.float32),
          pltpu.VMEM((_S_PER, ACT_D), jnp.float32),
          pltpu.SemaphoreType.DMA,
          pltpu.SemaphoreType.DMA,
          pltpu.SemaphoreType.DMA,
          pltpu.SemaphoreType.DMA,
          pltpu.SemaphoreType.DMA,
          pltpu.SemaphoreType.DMA,
          pltpu.SemaphoreType.DMA,
      ),
  )
  return k(word_ids1d, topic_ids, act_ids, over_w, over_t, over_a)


def _overlap(table, d):
  return jnp.concatenate([table] * (128 // d), axis=1)


def kernel(word_ids, topic_ids, act_ids, word_table, topic_table, act_table):
  word_ids1d = word_ids.reshape(_TOTAL_W).astype(jnp.int32)
  wout, tout, aout = _run(word_ids1d,
                          topic_ids.astype(jnp.int32),
                          act_ids.astype(jnp.int32),
                          _overlap(word_table, WORD_D),
                          _overlap(topic_table, TOPIC_D),
                          _overlap(act_table, ACT_D))
  return (wout.reshape(B, L, WORD_D), tout, aout)
